# baseline (speedup 1.0000x reference)
import functools

import jax
import jax.numpy as jnp
from jax import lax
from jax.experimental import pallas as pl
from jax.experimental.pallas import tpu as pltpu
from jax.experimental.pallas import tpu_sc as plsc

_B, _L, _LLIB, _T, _TLIB, _D = 16, 2048, 512, 16384, 4096, 256
_DC = 16
_NDC = _D // _DC
_NC, _NS = 2, 16
_NW = _NC * _NS
_TASKS = _B * _NDC
_TPW = _TASKS // _NW
_ECH = 2048
_NCH = _T // _ECH
_NCHL = _TLIB // _ECH
_NTOT = _NCH + _NCHL


def _build_sc_agg():
    mesh = plsc.VectorSubcoreMesh(
        core_axis_name="c", subcore_axis_name="s",
        num_cores=_NC, num_subcores=_NS)

    @functools.partial(
        pl.kernel,
        out_type=jax.ShapeDtypeStruct((_B, _D), jnp.float32),
        mesh=mesh,
        compiler_params=pltpu.CompilerParams(
            use_tc_tiling_on_sc=False, needs_layout_passes=False),
        scratch_types=[
            pltpu.VMEM((2, _L, _DC), jnp.float32),
            pltpu.VMEM((2, _LLIB, _DC), jnp.float32),
            pltpu.VMEM((_L, _DC), jnp.float32),
            pltpu.VMEM((2, 2, _ECH), jnp.int32),
            pltpu.VMEM((2, _ECH), jnp.float32),
            pltpu.VMEM((_DC,), jnp.float32),
            pltpu.SemaphoreType.DMA,
            pltpu.SemaphoreType.DMA,
            pltpu.SemaphoreType.DMA,
            pltpu.SemaphoreType.DMA,
            pltpu.SemaphoreType.DMA,
            pltpu.SemaphoreType.DMA,
        ],
    )
    def sc_agg(newembs, ind, values, newembs_lib, ind_lib, values_lib,
               out, table_v, tlib_v, acc_v, rc_v, vv_v, obuf_v,
               sem_rc0, sem_rc1, sem_vv0, sem_vv1, sem_tab0, sem_tab1):
        wid = lax.axis_index("s") * _NC + lax.axis_index("c")
        sem_rc = (sem_rc0, sem_rc1)
        sem_vv = (sem_vv0, sem_vv1)
        sem_tab = (sem_tab0, sem_tab1)

        def bd_of(tid):
            return tid // _NDC, (tid % _NDC) * _DC

        def issue_tables(tid, tbuf):
            b, dlo = bd_of(tid)
            for buf in (0, 1):
                @pl.when(tbuf == buf)
                def _():
                    pltpu.async_copy(
                        newembs.at[b, :, pl.ds(dlo, _DC)],
                        table_v.at[buf], sem_tab[buf])
                    pltpu.async_copy(
                        newembs_lib.at[b, :, pl.ds(dlo, _DC)],
                        tlib_v.at[buf], sem_tab[buf])

        def wait_tables(tid, tbuf):
            b, dlo = bd_of(tid)
            for buf in (0, 1):
                @pl.when(tbuf == buf)
                def _():
                    pltpu.make_async_copy(
                        newembs.at[b, :, pl.ds(dlo, _DC)],
                        table_v.at[buf], sem_tab[buf]).wait()
                    pltpu.make_async_copy(
                        newembs_lib.at[b, :, pl.ds(dlo, _DC)],
                        tlib_v.at[buf], sem_tab[buf]).wait()

        def zero_row(r):
            acc_v[r] = jnp.zeros((_DC,), jnp.float32)
        plsc.parallel_loop(0, _L)(zero_row)

        issue_tables(wid * _TPW, 0)

        def run_task(t, carry):
            tid = wid * _TPW + t
            b, dlo = bd_of(tid)
            tbuf = lax.rem(t, 2)
            wait_tables(tid, tbuf)

            @pl.when(t + 1 < _TPW)
            def _():
                issue_tables(tid + 1, 1 - tbuf)

            def issue(ci, buf):
                if ci < _NCH:
                    d1 = pltpu.async_copy(
                        ind.at[b, :, pl.ds(ci * _ECH, _ECH)],
                        rc_v.at[buf], sem_rc[buf])
                    d2 = pltpu.async_copy(
                        values.at[b, pl.ds(ci * _ECH, _ECH)],
                        vv_v.at[buf], sem_vv[buf])
                else:
                    cl = ci - _NCH
                    d1 = pltpu.async_copy(
                        ind_lib.at[b, :, pl.ds(cl * _ECH, _ECH)],
                        rc_v.at[buf], sem_rc[buf])
                    d2 = pltpu.async_copy(
                        values_lib.at[b, pl.ds(cl * _ECH, _ECH)],
                        vv_v.at[buf], sem_vv[buf])
                return d1, d2

            descs = [issue(0, 0), None]
            for ci in range(_NTOT):
                buf = ci % 2
                d1, d2 = descs[buf]
                d1.wait()
                d2.wait()
                if ci + 1 < _NTOT:
                    descs[(ci + 1) % 2] = issue(ci + 1, (ci + 1) % 2)
                tab2 = table_v if ci < _NCH else tlib_v

                def ent(g, buf=buf, tab2=tab2, tbuf=tbuf):
                    rows16 = rc_v[buf, 0, pl.ds(g * 16, 16)]
                    cols16 = rc_v[buf, 1, pl.ds(g * 16, 16)]
                    vals16 = vv_v[buf, pl.ds(g * 16, 16)]
                    for i in range(16):
                        emb = tab2[tbuf, cols16[i]]
                        plsc.addupdate(acc_v.at[rows16[i]],
                                       emb * vals16[i])
                plsc.parallel_loop(0, _ECH // 16, unroll=8)(ent)

            zero = jnp.zeros((_DC,), jnp.float32)

            def rsum(r, s):
                s0, s1, s2, s3 = s
                q = _L // 4
                v0 = acc_v[r]
                v1 = acc_v[r + q]
                v2 = acc_v[r + 2 * q]
                v3 = acc_v[r + 3 * q]
                acc_v[r] = zero
                acc_v[r + q] = zero
                acc_v[r + 2 * q] = zero
                acc_v[r + 3 * q] = zero
                return (s0 + jnp.maximum(v0, 0.0),
                        s1 + jnp.maximum(v1, 0.0),
                        s2 + jnp.maximum(v2, 0.0),
                        s3 + jnp.maximum(v3, 0.0))
            s0, s1, s2, s3 = lax.fori_loop(0, _L // 4, rsum,
                                           (zero, zero, zero, zero))
            obuf_v[...] = (s0 + s1) + (s2 + s3)
            pltpu.sync_copy(obuf_v, out.at[b, pl.ds(dlo, _DC)])
            return carry

        lax.fori_loop(0, _TPW, run_task, 0)

    return sc_agg


_SC_AGG_CACHE = []


def _sc_agg_fn():
    if not _SC_AGG_CACHE:
        _SC_AGG_CACHE.append(_build_sc_agg())
    return _SC_AGG_CACHE[0]


def _finalize_body(p_ref, m_ref, w_ref, q_ref, o_ref):
    m = m_ref[...]
    denom = jnp.clip(jnp.sum(m, axis=1, keepdims=True), 1.0, None)
    pooled = p_ref[...] / denom
    target = jnp.dot(pooled.astype(jnp.bfloat16),
                     w_ref[...].astype(jnp.bfloat16),
                     preferred_element_type=jnp.float32)
    q = q_ref[0, :]
    qn = q / (jnp.sqrt(jnp.sum(q * q)) + 1e-8)
    tnorm = jnp.sqrt(jnp.sum(target * target, axis=1, keepdims=True)) + 1e-8
    tn = target / tnorm
    scores = jnp.dot(tn.astype(jnp.bfloat16),
                     qn.astype(jnp.bfloat16).reshape(_D, 1),
                     preferred_element_type=jnp.float32)
    o_ref[...] = jnp.broadcast_to(scores, (_B, 128))


def kernel(newembs, values, newembs_lib, values_lib, masks, W, query_emb,
           ind, ind_lib):
    pooled_sums = _sc_agg_fn()(newembs, ind.astype(jnp.int32), values,
                               newembs_lib, ind_lib.astype(jnp.int32),
                               values_lib)
    scores128 = pl.pallas_call(
        _finalize_body,
        out_shape=jax.ShapeDtypeStruct((_B, 128), jnp.float32),
    )(pooled_sums, masks, W, query_emb.reshape(1, _D))
    return scores128[:, 0]


# speedup vs baseline: 1.5087x; 1.5087x over previous
import functools

import jax
import jax.numpy as jnp
from jax import lax
from jax.experimental import pallas as pl
from jax.experimental.pallas import tpu as pltpu
from jax.experimental.pallas import tpu_sc as plsc

_B, _L, _LLIB, _T, _TLIB, _D = 16, 2048, 512, 16384, 4096, 256
_DC = 16
_NDC = _D // _DC
_NC, _NS = 2, 16
_NW = _NC * _NS
_TASKS = _B * _NDC
_TPW = _TASKS // _NW
_ECH = 2048
_NCH = _T // _ECH
_NCHL = _TLIB // _ECH
_NTOT = _NCH + _NCHL


def _build_sc_agg():
    mesh = plsc.VectorSubcoreMesh(
        core_axis_name="c", subcore_axis_name="s",
        num_cores=_NC, num_subcores=_NS)

    @functools.partial(
        pl.kernel,
        out_type=jax.ShapeDtypeStruct((_B, _D), jnp.float32),
        mesh=mesh,
        compiler_params=pltpu.CompilerParams(
            use_tc_tiling_on_sc=False, needs_layout_passes=False),
        scratch_types=[
            pltpu.VMEM((2, _L, _DC), jnp.float32),
            pltpu.VMEM((2, _LLIB, _DC), jnp.float32),
            pltpu.VMEM((_L, _DC), jnp.float32),
            pltpu.VMEM((2, 2, _ECH), jnp.int32),
            pltpu.VMEM((2, _ECH), jnp.float32),
            pltpu.VMEM((_DC,), jnp.float32),
            pltpu.SemaphoreType.DMA,
            pltpu.SemaphoreType.DMA,
            pltpu.SemaphoreType.DMA,
            pltpu.SemaphoreType.DMA,
            pltpu.SemaphoreType.DMA,
            pltpu.SemaphoreType.DMA,
        ],
    )
    def sc_agg(newembs, ind, values, newembs_lib, ind_lib, values_lib,
               out, table_v, tlib_v, acc_v, rc_v, vv_v, obuf_v,
               sem_rc0, sem_rc1, sem_vv0, sem_vv1, sem_tab0, sem_tab1):
        wid = lax.axis_index("s") * _NC + lax.axis_index("c")
        sem_rc = (sem_rc0, sem_rc1)
        sem_vv = (sem_vv0, sem_vv1)
        sem_tab = (sem_tab0, sem_tab1)

        def bd_of(tid):
            return tid // _NDC, (tid % _NDC) * _DC

        def issue_tables(tid, tbuf):
            b, dlo = bd_of(tid)
            for buf in (0, 1):
                @pl.when(tbuf == buf)
                def _():
                    pltpu.async_copy(
                        newembs.at[b, :, pl.ds(dlo, _DC)],
                        table_v.at[buf], sem_tab[buf])
                    pltpu.async_copy(
                        newembs_lib.at[b, :, pl.ds(dlo, _DC)],
                        tlib_v.at[buf], sem_tab[buf])

        def wait_tables(tid, tbuf):
            b, dlo = bd_of(tid)
            for buf in (0, 1):
                @pl.when(tbuf == buf)
                def _():
                    pltpu.make_async_copy(
                        newembs.at[b, :, pl.ds(dlo, _DC)],
                        table_v.at[buf], sem_tab[buf]).wait()
                    pltpu.make_async_copy(
                        newembs_lib.at[b, :, pl.ds(dlo, _DC)],
                        tlib_v.at[buf], sem_tab[buf]).wait()

        def zero_row(r):
            acc_v[r] = jnp.zeros((_DC,), jnp.float32)
        plsc.parallel_loop(0, _L)(zero_row)

        issue_tables(wid * _TPW, 0)

        def run_task(t, carry):
            tid = wid * _TPW + t
            b, dlo = bd_of(tid)
            tbuf = lax.rem(t, 2)
            wait_tables(tid, tbuf)

            @pl.when(t + 1 < _TPW)
            def _():
                issue_tables(tid + 1, 1 - tbuf)

            def issue(ci, buf):
                if ci < _NCH:
                    d1 = pltpu.async_copy(
                        ind.at[b, :, pl.ds(ci * _ECH, _ECH)],
                        rc_v.at[buf], sem_rc[buf])
                    d2 = pltpu.async_copy(
                        values.at[b, pl.ds(ci * _ECH, _ECH)],
                        vv_v.at[buf], sem_vv[buf])
                else:
                    cl = ci - _NCH
                    d1 = pltpu.async_copy(
                        ind_lib.at[b, :, pl.ds(cl * _ECH, _ECH)],
                        rc_v.at[buf], sem_rc[buf])
                    d2 = pltpu.async_copy(
                        values_lib.at[b, pl.ds(cl * _ECH, _ECH)],
                        vv_v.at[buf], sem_vv[buf])
                return d1, d2

            descs = [issue(0, 0), None]
            for ci in range(_NTOT):
                buf = ci % 2
                d1, d2 = descs[buf]
                d1.wait()
                d2.wait()
                if ci + 1 < _NTOT:
                    descs[(ci + 1) % 2] = issue(ci + 1, (ci + 1) % 2)
                tab2 = table_v if ci < _NCH else tlib_v

                def ent(g, buf=buf, tab2=tab2, tbuf=tbuf):
                    rows16 = rc_v[buf, 0, pl.ds(g * 16, 16)]
                    cols16 = rc_v[buf, 1, pl.ds(g * 16, 16)]
                    vals16 = vv_v[buf, pl.ds(g * 16, 16)]
                    for i in range(16):
                        emb = tab2[tbuf, cols16[i]]
                        plsc.addupdate(acc_v.at[rows16[i]],
                                       emb * vals16[i])
                plsc.parallel_loop(0, _ECH // 16, unroll=2)(ent)

            zero = jnp.zeros((_DC,), jnp.float32)

            def rsum(r, s):
                s0, s1, s2, s3 = s
                q = _L // 4
                v0 = acc_v[r]
                v1 = acc_v[r + q]
                v2 = acc_v[r + 2 * q]
                v3 = acc_v[r + 3 * q]
                acc_v[r] = zero
                acc_v[r + q] = zero
                acc_v[r + 2 * q] = zero
                acc_v[r + 3 * q] = zero
                return (s0 + jnp.maximum(v0, 0.0),
                        s1 + jnp.maximum(v1, 0.0),
                        s2 + jnp.maximum(v2, 0.0),
                        s3 + jnp.maximum(v3, 0.0))
            s0, s1, s2, s3 = lax.fori_loop(0, _L // 4, rsum,
                                           (zero, zero, zero, zero))
            obuf_v[...] = (s0 + s1) + (s2 + s3)
            pltpu.sync_copy(obuf_v, out.at[b, pl.ds(dlo, _DC)])
            return carry

        lax.fori_loop(0, _TPW, run_task, 0)

    return sc_agg


_SC_AGG_CACHE = []


def _sc_agg_fn():
    if not _SC_AGG_CACHE:
        _SC_AGG_CACHE.append(_build_sc_agg())
    return _SC_AGG_CACHE[0]


def _finalize_body(p_ref, m_ref, w_ref, q_ref, o_ref):
    m = m_ref[...]
    denom = jnp.clip(jnp.sum(m, axis=1, keepdims=True), 1.0, None)
    pooled = p_ref[...] / denom
    target = jnp.dot(pooled.astype(jnp.bfloat16),
                     w_ref[...].astype(jnp.bfloat16),
                     preferred_element_type=jnp.float32)
    q = q_ref[0, :]
    qn = q / (jnp.sqrt(jnp.sum(q * q)) + 1e-8)
    tnorm = jnp.sqrt(jnp.sum(target * target, axis=1, keepdims=True)) + 1e-8
    tn = target / tnorm
    scores = jnp.dot(tn.astype(jnp.bfloat16),
                     qn.astype(jnp.bfloat16).reshape(_D, 1),
                     preferred_element_type=jnp.float32)
    o_ref[...] = jnp.broadcast_to(scores, (_B, 128))


def kernel(newembs, values, newembs_lib, values_lib, masks, W, query_emb,
           ind, ind_lib):
    pooled_sums = _sc_agg_fn()(newembs, ind.astype(jnp.int32), values,
                               newembs_lib, ind_lib.astype(jnp.int32),
                               values_lib)
    scores128 = pl.pallas_call(
        _finalize_body,
        out_shape=jax.ShapeDtypeStruct((_B, 128), jnp.float32),
    )(pooled_sums, masks, W, query_emb.reshape(1, _D))
    return scores128[:, 0]


# speedup vs baseline: 1.5472x; 1.0255x over previous
import functools

import jax
import jax.numpy as jnp
from jax import lax
from jax.experimental import pallas as pl
from jax.experimental.pallas import tpu as pltpu
from jax.experimental.pallas import tpu_sc as plsc

_B, _L, _LLIB, _T, _TLIB, _D = 16, 2048, 512, 16384, 4096, 256
_DC = 16
_NDC = _D // _DC
_NC, _NS = 2, 16
_NW = _NC * _NS
_TASKS = _B * _NDC
_TPW = _TASKS // _NW
_ECH = 2048
_NCH = _T // _ECH
_NCHL = _TLIB // _ECH
_NTOT = _NCH + _NCHL


def _build_sc_agg():
    mesh = plsc.VectorSubcoreMesh(
        core_axis_name="c", subcore_axis_name="s",
        num_cores=_NC, num_subcores=_NS)

    @functools.partial(
        pl.kernel,
        out_type=jax.ShapeDtypeStruct((_B, _D), jnp.float32),
        mesh=mesh,
        compiler_params=pltpu.CompilerParams(
            use_tc_tiling_on_sc=False, needs_layout_passes=False),
        scratch_types=[
            pltpu.VMEM((2, _L, _DC), jnp.float32),
            pltpu.VMEM((2, _LLIB, _DC), jnp.float32),
            pltpu.VMEM((_L, _DC), jnp.float32),
            pltpu.VMEM((2, 2, _ECH), jnp.int32),
            pltpu.VMEM((2, _ECH), jnp.float32),
            pltpu.VMEM((_DC,), jnp.float32),
            pltpu.SemaphoreType.DMA,
            pltpu.SemaphoreType.DMA,
            pltpu.SemaphoreType.DMA,
            pltpu.SemaphoreType.DMA,
            pltpu.SemaphoreType.DMA,
            pltpu.SemaphoreType.DMA,
        ],
    )
    def sc_agg(newembs, ind, values, newembs_lib, ind_lib, values_lib,
               out, table_v, tlib_v, acc_v, rc_v, vv_v, obuf_v,
               sem_rc0, sem_rc1, sem_vv0, sem_vv1, sem_tab0, sem_tab1):
        wid = lax.axis_index("s") * _NC + lax.axis_index("c")
        sem_rc = (sem_rc0, sem_rc1)
        sem_vv = (sem_vv0, sem_vv1)
        sem_tab = (sem_tab0, sem_tab1)

        def bd_of(tid):
            return tid // _NDC, (tid % _NDC) * _DC

        def issue_tables(tid, tbuf):
            b, dlo = bd_of(tid)
            for buf in (0, 1):
                @pl.when(tbuf == buf)
                def _():
                    pltpu.async_copy(
                        newembs.at[b, :, pl.ds(dlo, _DC)],
                        table_v.at[buf], sem_tab[buf])
                    pltpu.async_copy(
                        newembs_lib.at[b, :, pl.ds(dlo, _DC)],
                        tlib_v.at[buf], sem_tab[buf])

        def wait_tables(tid, tbuf):
            b, dlo = bd_of(tid)
            for buf in (0, 1):
                @pl.when(tbuf == buf)
                def _():
                    pltpu.make_async_copy(
                        newembs.at[b, :, pl.ds(dlo, _DC)],
                        table_v.at[buf], sem_tab[buf]).wait()
                    pltpu.make_async_copy(
                        newembs_lib.at[b, :, pl.ds(dlo, _DC)],
                        tlib_v.at[buf], sem_tab[buf]).wait()

        def zero_row(r):
            acc_v[r] = jnp.zeros((_DC,), jnp.float32)
        plsc.parallel_loop(0, _L)(zero_row)

        issue_tables(wid * _TPW, 0)
        b00 = (wid * _TPW) // _NDC
        pltpu.async_copy(ind.at[b00, :, pl.ds(0, _ECH)], rc_v.at[0],
                         sem_rc[0])
        pltpu.async_copy(values.at[b00, pl.ds(0, _ECH)], vv_v.at[0],
                         sem_vv[0])

        def run_task(t, carry):
            tid = wid * _TPW + t
            b, dlo = bd_of(tid)
            tbuf = lax.rem(t, 2)
            wait_tables(tid, tbuf)

            @pl.when(t + 1 < _TPW)
            def _():
                issue_tables(tid + 1, 1 - tbuf)

            def issue(ci, buf, bb):
                if ci < _NCH:
                    d1 = pltpu.async_copy(
                        ind.at[bb, :, pl.ds(ci * _ECH, _ECH)],
                        rc_v.at[buf], sem_rc[buf])
                    d2 = pltpu.async_copy(
                        values.at[bb, pl.ds(ci * _ECH, _ECH)],
                        vv_v.at[buf], sem_vv[buf])
                else:
                    cl = ci - _NCH
                    d1 = pltpu.async_copy(
                        ind_lib.at[bb, :, pl.ds(cl * _ECH, _ECH)],
                        rc_v.at[buf], sem_rc[buf])
                    d2 = pltpu.async_copy(
                        values_lib.at[bb, pl.ds(cl * _ECH, _ECH)],
                        vv_v.at[buf], sem_vv[buf])
                return d1, d2

            descs = [
                (pltpu.make_async_copy(ind.at[b, :, pl.ds(0, _ECH)],
                                       rc_v.at[0], sem_rc[0]),
                 pltpu.make_async_copy(values.at[b, pl.ds(0, _ECH)],
                                       vv_v.at[0], sem_vv[0])),
                None,
            ]
            for ci in range(_NTOT):
                buf = ci % 2
                d1, d2 = descs[buf]
                d1.wait()
                d2.wait()
                if ci + 1 < _NTOT:
                    descs[(ci + 1) % 2] = issue(ci + 1, (ci + 1) % 2, b)
                tab2 = table_v if ci < _NCH else tlib_v
                if ci == _NTOT - 1:
                    @pl.when(t + 1 < _TPW)
                    def _():
                        bn = (tid + 1) // _NDC
                        pltpu.async_copy(
                            ind.at[bn, :, pl.ds(0, _ECH)],
                            rc_v.at[0], sem_rc[0])
                        pltpu.async_copy(
                            values.at[bn, pl.ds(0, _ECH)],
                            vv_v.at[0], sem_vv[0])

                def ent(g, buf=buf, tab2=tab2, tbuf=tbuf):
                    rows16 = rc_v[buf, 0, pl.ds(g * 16, 16)]
                    cols16 = rc_v[buf, 1, pl.ds(g * 16, 16)]
                    vals16 = vv_v[buf, pl.ds(g * 16, 16)]
                    for i in range(16):
                        emb = tab2[tbuf, cols16[i]]
                        plsc.addupdate(acc_v.at[rows16[i]],
                                       emb * vals16[i])
                plsc.parallel_loop(0, _ECH // 16, unroll=2)(ent)

            zero = jnp.zeros((_DC,), jnp.float32)

            def rsum(r, s):
                s0, s1, s2, s3 = s
                q = _L // 4
                v0 = acc_v[r]
                v1 = acc_v[r + q]
                v2 = acc_v[r + 2 * q]
                v3 = acc_v[r + 3 * q]
                acc_v[r] = zero
                acc_v[r + q] = zero
                acc_v[r + 2 * q] = zero
                acc_v[r + 3 * q] = zero
                return (s0 + jnp.maximum(v0, 0.0),
                        s1 + jnp.maximum(v1, 0.0),
                        s2 + jnp.maximum(v2, 0.0),
                        s3 + jnp.maximum(v3, 0.0))
            s0, s1, s2, s3 = lax.fori_loop(0, _L // 4, rsum,
                                           (zero, zero, zero, zero))
            obuf_v[...] = (s0 + s1) + (s2 + s3)
            pltpu.sync_copy(obuf_v, out.at[b, pl.ds(dlo, _DC)])
            return carry

        lax.fori_loop(0, _TPW, run_task, 0)

    return sc_agg


_SC_AGG_CACHE = []


def _sc_agg_fn():
    if not _SC_AGG_CACHE:
        _SC_AGG_CACHE.append(_build_sc_agg())
    return _SC_AGG_CACHE[0]


def _finalize_body(p_ref, m_ref, w_ref, q_ref, o_ref):
    m = m_ref[...]
    denom = jnp.clip(jnp.sum(m, axis=1, keepdims=True), 1.0, None)
    pooled = p_ref[...] / denom
    target = jnp.dot(pooled.astype(jnp.bfloat16),
                     w_ref[...].astype(jnp.bfloat16),
                     preferred_element_type=jnp.float32)
    q = q_ref[0, :]
    qn = q / (jnp.sqrt(jnp.sum(q * q)) + 1e-8)
    tnorm = jnp.sqrt(jnp.sum(target * target, axis=1, keepdims=True)) + 1e-8
    tn = target / tnorm
    scores = jnp.dot(tn.astype(jnp.bfloat16),
                     qn.astype(jnp.bfloat16).reshape(_D, 1),
                     preferred_element_type=jnp.float32)
    o_ref[...] = jnp.broadcast_to(scores, (_B, 128))


def kernel(newembs, values, newembs_lib, values_lib, masks, W, query_emb,
           ind, ind_lib):
    pooled_sums = _sc_agg_fn()(newembs, ind.astype(jnp.int32), values,
                               newembs_lib, ind_lib.astype(jnp.int32),
                               values_lib)
    scores128 = pl.pallas_call(
        _finalize_body,
        out_shape=jax.ShapeDtypeStruct((_B, 128), jnp.float32),
    )(pooled_sums, masks, W, query_emb.reshape(1, _D))
    return scores128[:, 0]
